# Initial kernel scaffold; baseline (speedup 1.0000x reference)
import jax
import jax.numpy as jnp
from jax.experimental import pallas as pl


def kernel(x, edge_index, W_l1, b_l1, W_r1, W_l2, b_l2, W_r2, W_l3, b_l3, W_r3):
    raise NotImplementedError("write your pallas kernel here")


# speedup vs baseline: 5.1011x; 5.1011x over previous
import functools

import jax
import jax.numpy as jnp
from jax import lax
from jax.experimental import pallas as pl
from jax.experimental.pallas import tpu as pltpu
from jax.experimental.pallas import tpu_sc as plsc

_N = 10000
_E = 20000
_D = 4096
_H = 256
_C = 6
_EP = 20480
_NP = 10112
_RB = 400

_ROWS_PER_TILE = _NP // 16
_CHUNKS = _EP // (16 * 128)


def _sc_agg_body(y_hbm, src_hbm, dst_hbm, agg_hbm,
                 sh_agg, src_v, dst_v, gidx, sidx, rows_v, sem):
    c = lax.axis_index("c")
    s = lax.axis_index("s")

    pltpu.sync_copy(src_hbm.at[s], src_v)
    pltpu.sync_copy(dst_hbm.at[s], dst_v)

    zero16 = jnp.zeros((16,), jnp.float32)

    def _zb(i, carry):
        rows_v[i // 8, pl.ds((i % 8) * 16, 16)] = zero16
        return carry
    lax.fori_loop(0, 1024, _zb, 0)

    base = s * _ROWS_PER_TILE
    for kk in range(4):
        pltpu.sync_copy(rows_v, sh_agg.at[pl.ds(base + kk * 128, 128)])
    pltpu.sync_copy(rows_v.at[pl.ds(0, _ROWS_PER_TILE - 512)],
                    sh_agg.at[pl.ds(base + 512, _ROWS_PER_TILE - 512)])

    plsc.subcore_barrier()

    off = c * _N

    def _chunk(k, carry):
        for j in range(8):
            sl = pl.ds(j * 16, 16)
            gidx[sl] = src_v[k, sl] + off
            sidx[sl] = dst_v[k, sl]
        pltpu.async_copy(y_hbm.at[gidx], rows_v, sem).wait()
        pltpu.sync_copy(rows_v, sh_agg.at[sidx], add=True)
        return carry
    lax.fori_loop(0, _CHUNKS, _chunk, 0)

    plsc.subcore_barrier()

    pltpu.sync_copy(sh_agg.at[pl.ds(base, _ROWS_PER_TILE)],
                    agg_hbm.at[c, pl.ds(base, _ROWS_PER_TILE)])


def _make_sc_agg():
    mesh = plsc.VectorSubcoreMesh(core_axis_name="c", subcore_axis_name="s",
                                  num_cores=2, num_subcores=16)
    return pl.kernel(
        _sc_agg_body,
        out_type=jax.ShapeDtypeStruct((2, _NP, 128), jnp.float32),
        mesh=mesh,
        scratch_types=[
            pltpu.VMEM_SHARED((_NP, 128), jnp.float32),
            pltpu.VMEM((_CHUNKS, 128), jnp.int32),
            pltpu.VMEM((_CHUNKS, 128), jnp.int32),
            pltpu.VMEM((128,), jnp.int32),
            pltpu.VMEM((128,), jnp.int32),
            pltpu.VMEM((128, 128), jnp.float32),
            pltpu.SemaphoreType.DMA,
        ],
    )


def _sc_cnt_body(dst_hbm, cnt_hbm, sh_cnt, dst_v, sidx, rows_v):
    c = lax.axis_index("c")
    s = lax.axis_index("s")

    pltpu.sync_copy(dst_hbm.at[s], dst_v)

    zero16 = jnp.zeros((16,), jnp.float32)

    def _zb(i, carry):
        rows_v[i // 8, pl.ds((i % 8) * 16, 16)] = zero16
        return carry
    lax.fori_loop(0, 1024, _zb, 0)

    base = s * _ROWS_PER_TILE
    for kk in range(4):
        pltpu.sync_copy(rows_v, sh_cnt.at[pl.ds(base + kk * 128, 128)])
    pltpu.sync_copy(rows_v.at[pl.ds(0, _ROWS_PER_TILE - 512)],
                    sh_cnt.at[pl.ds(base + 512, _ROWS_PER_TILE - 512)])

    one16 = jnp.ones((16,), jnp.float32)

    def _ob(i, carry):
        rows_v[i // 8, pl.ds((i % 8) * 16, 16)] = one16
        return carry
    lax.fori_loop(0, 1024, _ob, 0)

    plsc.subcore_barrier()

    def _chunk(k, carry):
        for j in range(8):
            sl = pl.ds(j * 16, 16)
            sidx[sl] = dst_v[k, sl]
        pltpu.sync_copy(rows_v, sh_cnt.at[sidx], add=True)
        return carry
    lax.fori_loop(0, _CHUNKS, _chunk, 0)

    plsc.subcore_barrier()

    @pl.when(c == 0)
    def _():
        pltpu.sync_copy(sh_cnt.at[pl.ds(base, _ROWS_PER_TILE)],
                        cnt_hbm.at[pl.ds(base, _ROWS_PER_TILE)])


def _make_sc_cnt():
    mesh = plsc.VectorSubcoreMesh(core_axis_name="c", subcore_axis_name="s",
                                  num_cores=2, num_subcores=16)
    return pl.kernel(
        _sc_cnt_body,
        out_type=jax.ShapeDtypeStruct((_NP, 128), jnp.float32),
        mesh=mesh,
        scratch_types=[
            pltpu.VMEM_SHARED((_NP, 128), jnp.float32),
            pltpu.VMEM((_CHUNKS, 128), jnp.int32),
            pltpu.VMEM((128,), jnp.int32),
            pltpu.VMEM((128, 128), jnp.float32),
        ],
    )


def _mm1(x, wcat):
    def body(x_ref, w_ref, yl_ref, yr_ref):
        y = jnp.dot(x_ref[...], w_ref[...],
                    preferred_element_type=jnp.float32)
        yl_ref[0] = y[:, :128]
        yl_ref[1] = y[:, 128:256]
        yr_ref[...] = y[:, 256:]

    return pl.pallas_call(
        body,
        grid=(_N // _RB,),
        in_specs=[
            pl.BlockSpec((_RB, _D), lambda i: (i, 0)),
            pl.BlockSpec((_D, 2 * _H), lambda i: (0, 0)),
        ],
        out_specs=[
            pl.BlockSpec((2, _RB, 128), lambda i: (0, i, 0)),
            pl.BlockSpec((_RB, _H), lambda i: (i, 0)),
        ],
        out_shape=[
            jax.ShapeDtypeStruct((2, _N, 128), jnp.float32),
            jax.ShapeDtypeStruct((_N, _H), jnp.float32),
        ],
    )(x, wcat)


def _combine_mm(agg, cnt, yr, b, wcat):
    def body(agg_ref, cnt_ref, yr_ref, b_ref, w_ref, yl_o, yr_o):
        cv = jnp.maximum(cnt_ref[:, 0:1], 1.0)
        mean = jnp.concatenate([agg_ref[0], agg_ref[1]], axis=1) / cv
        h = jnp.maximum(mean + b_ref[...] + yr_ref[...], 0.0)
        y = jnp.dot(h, w_ref[...], preferred_element_type=jnp.float32)
        yl_o[0] = y[:, :128]
        yl_o[1] = y[:, 128:256]
        yr_o[...] = y[:, 256:]

    return pl.pallas_call(
        body,
        grid=(_N // _RB,),
        in_specs=[
            pl.BlockSpec((2, _RB, 128), lambda i: (0, i, 0)),
            pl.BlockSpec((_RB, 128), lambda i: (i, 0)),
            pl.BlockSpec((_RB, _H), lambda i: (i, 0)),
            pl.BlockSpec((1, _H), lambda i: (0, 0)),
            pl.BlockSpec((_H, 2 * _H), lambda i: (0, 0)),
        ],
        out_specs=[
            pl.BlockSpec((2, _RB, 128), lambda i: (0, i, 0)),
            pl.BlockSpec((_RB, _H), lambda i: (i, 0)),
        ],
        out_shape=[
            jax.ShapeDtypeStruct((2, _N, 128), jnp.float32),
            jax.ShapeDtypeStruct((_N, _H), jnp.float32),
        ],
    )(agg, cnt, yr, b, wcat)


def _combine_only(agg, cnt, yr, b):
    def body(agg_ref, cnt_ref, yr_ref, b_ref, h_o, hblk_o):
        cv = jnp.maximum(cnt_ref[:, 0:1], 1.0)
        mean = jnp.concatenate([agg_ref[0], agg_ref[1]], axis=1) / cv
        h = jnp.maximum(mean + b_ref[...] + yr_ref[...], 0.0)
        h_o[...] = h
        hblk_o[0] = h[:, :128]
        hblk_o[1] = h[:, 128:256]

    return pl.pallas_call(
        body,
        grid=(_N // _RB,),
        in_specs=[
            pl.BlockSpec((2, _RB, 128), lambda i: (0, i, 0)),
            pl.BlockSpec((_RB, 128), lambda i: (i, 0)),
            pl.BlockSpec((_RB, _H), lambda i: (i, 0)),
            pl.BlockSpec((1, _H), lambda i: (0, 0)),
        ],
        out_specs=[
            pl.BlockSpec((_RB, _H), lambda i: (i, 0)),
            pl.BlockSpec((2, _RB, 128), lambda i: (0, i, 0)),
        ],
        out_shape=[
            jax.ShapeDtypeStruct((_N, _H), jnp.float32),
            jax.ShapeDtypeStruct((2, _N, 128), jnp.float32),
        ],
    )(agg, cnt, yr, b)


def _final(agg, cnt, h, wl, wr, b):
    def body(agg_ref, cnt_ref, h_ref, wl_ref, wr_ref, b_ref, o_ref):
        cv = jnp.maximum(cnt_ref[:, 0:1], 1.0)
        mean = jnp.concatenate([agg_ref[0], agg_ref[1]], axis=1) / cv
        o_ref[...] = (
            jnp.dot(mean, wl_ref[...], preferred_element_type=jnp.float32)
            + b_ref[...]
            + jnp.dot(h_ref[...], wr_ref[...],
                      preferred_element_type=jnp.float32))

    return pl.pallas_call(
        body,
        grid=(_N // _RB,),
        in_specs=[
            pl.BlockSpec((2, _RB, 128), lambda i: (0, i, 0)),
            pl.BlockSpec((_RB, 128), lambda i: (i, 0)),
            pl.BlockSpec((_RB, _H), lambda i: (i, 0)),
            pl.BlockSpec((_H, _C), lambda i: (0, 0)),
            pl.BlockSpec((_H, _C), lambda i: (0, 0)),
            pl.BlockSpec((1, _C), lambda i: (0, 0)),
        ],
        out_specs=pl.BlockSpec((_RB, _C), lambda i: (i, 0)),
        out_shape=jax.ShapeDtypeStruct((_N, _C), jnp.float32),
    )(agg, cnt, h, wl, wr, b)


_sc_agg = _make_sc_agg()
_sc_cnt = _make_sc_cnt()


def kernel(x, edge_index, W_l1, b_l1, W_r1, W_l2, b_l2, W_r2,
           W_l3, b_l3, W_r3):
    src = edge_index[0]
    dst = edge_index[1]
    pad = _EP - _E
    srcp = jnp.concatenate(
        [src, jnp.zeros((pad,), jnp.int32)]).reshape(16, _CHUNKS, 128)
    dstp = jnp.concatenate(
        [dst, jnp.full((pad,), _N, jnp.int32)]).reshape(16, _CHUNKS, 128)

    wcat1 = jnp.concatenate([W_l1, W_r1], axis=0).T
    wcat2 = jnp.concatenate([W_l2, W_r2], axis=0).T

    y1l, y1r = _mm1(x, wcat1)
    cntp = _sc_cnt(dstp)
    agg1 = _sc_agg(y1l.reshape(2 * _N, 128), srcp, dstp)
    y2l, y2r = _combine_mm(agg1, cntp, y1r, b_l1.reshape(1, _H), wcat2)
    agg2 = _sc_agg(y2l.reshape(2 * _N, 128), srcp, dstp)
    h2, h2blk = _combine_only(agg2, cntp, y2r, b_l2.reshape(1, _H))
    agg3 = _sc_agg(h2blk.reshape(2 * _N, 128), srcp, dstp)
    out = _final(agg3, cntp, h2, W_l3.T, W_r3.T, b_l3.reshape(1, _C))
    return (h2, out)


# speedup vs baseline: 5.3788x; 1.0544x over previous
import functools

import jax
import jax.numpy as jnp
from jax import lax
from jax.experimental import pallas as pl
from jax.experimental.pallas import tpu as pltpu
from jax.experimental.pallas import tpu_sc as plsc

_N = 10000
_E = 20000
_D = 4096
_H = 256
_C = 6
_EP = 20480
_NP = 10112
_RB = 400

_ROWS_PER_TILE = _NP // 16
_CHUNKS = _EP // (16 * 128)


def _sc_agg_body(y_hbm, src_hbm, dst_hbm, agg_hbm,
                 sh_agg, src_v, dst_v, gidx, sidx, rows_v, rows2_v,
                 sem, sem2):
    c = lax.axis_index("c")
    s = lax.axis_index("s")

    pltpu.sync_copy(src_hbm.at[s], src_v)
    pltpu.sync_copy(dst_hbm.at[s], dst_v)

    zero16 = jnp.zeros((16,), jnp.float32)

    def _zb(i, carry):
        rows_v[i // 8, pl.ds((i % 8) * 16, 16)] = zero16
        return carry
    lax.fori_loop(0, 1024, _zb, 0)

    base = s * _ROWS_PER_TILE
    for kk in range(4):
        pltpu.sync_copy(rows_v, sh_agg.at[pl.ds(base + kk * 128, 128)])
    pltpu.sync_copy(rows_v.at[pl.ds(0, _ROWS_PER_TILE - 512)],
                    sh_agg.at[pl.ds(base + 512, _ROWS_PER_TILE - 512)])

    plsc.subcore_barrier()

    off = c * _N

    bufs = (rows_v, rows2_v)
    sems = (sem, sem2)

    def _fill_gidx(k):
        for j in range(8):
            sl = pl.ds(j * 16, 16)
            gidx[sl] = src_v[k, sl] + off

    _fill_gidx(0)
    desc = pltpu.async_copy(y_hbm.at[gidx], bufs[0], sems[0])
    for k in range(_CHUNKS):
        desc.wait()
        if k + 1 < _CHUNKS:
            _fill_gidx(k + 1)
            desc = pltpu.async_copy(y_hbm.at[gidx], bufs[(k + 1) % 2],
                                    sems[(k + 1) % 2])
        for j in range(8):
            sl = pl.ds(j * 16, 16)
            sidx[sl] = dst_v[k, sl]
        pltpu.sync_copy(bufs[k % 2], sh_agg.at[sidx], add=True)

    plsc.subcore_barrier()

    pltpu.sync_copy(sh_agg.at[pl.ds(base, _ROWS_PER_TILE)],
                    agg_hbm.at[c, pl.ds(base, _ROWS_PER_TILE)])


def _make_sc_agg():
    mesh = plsc.VectorSubcoreMesh(core_axis_name="c", subcore_axis_name="s",
                                  num_cores=2, num_subcores=16)
    return pl.kernel(
        _sc_agg_body,
        out_type=jax.ShapeDtypeStruct((2, _NP, 128), jnp.float32),
        mesh=mesh,
        scratch_types=[
            pltpu.VMEM_SHARED((_NP, 128), jnp.float32),
            pltpu.VMEM((_CHUNKS, 128), jnp.int32),
            pltpu.VMEM((_CHUNKS, 128), jnp.int32),
            pltpu.VMEM((128,), jnp.int32),
            pltpu.VMEM((128,), jnp.int32),
            pltpu.VMEM((128, 128), jnp.float32),
            pltpu.VMEM((128, 128), jnp.float32),
            pltpu.SemaphoreType.DMA,
            pltpu.SemaphoreType.DMA,
        ],
    )


def _sc_cnt_body(dst_hbm, cnt_hbm, sh_cnt, dst_v, sidx, rows_v):
    c = lax.axis_index("c")
    s = lax.axis_index("s")

    pltpu.sync_copy(dst_hbm.at[s], dst_v)

    zero16 = jnp.zeros((16,), jnp.float32)

    def _zb(i, carry):
        rows_v[i // 8, pl.ds((i % 8) * 16, 16)] = zero16
        return carry
    lax.fori_loop(0, 1024, _zb, 0)

    base = s * _ROWS_PER_TILE
    for kk in range(4):
        pltpu.sync_copy(rows_v, sh_cnt.at[pl.ds(base + kk * 128, 128)])
    pltpu.sync_copy(rows_v.at[pl.ds(0, _ROWS_PER_TILE - 512)],
                    sh_cnt.at[pl.ds(base + 512, _ROWS_PER_TILE - 512)])

    one16 = jnp.ones((16,), jnp.float32)

    def _ob(i, carry):
        rows_v[i // 8, pl.ds((i % 8) * 16, 16)] = one16
        return carry
    lax.fori_loop(0, 1024, _ob, 0)

    plsc.subcore_barrier()

    half = _CHUNKS // 2

    def _chunk(k, carry):
        kk = c * half + k
        for j in range(8):
            sl = pl.ds(j * 16, 16)
            sidx[sl] = dst_v[kk, sl]
        pltpu.sync_copy(rows_v, sh_cnt.at[sidx], add=True)
        return carry
    lax.fori_loop(0, half, _chunk, 0)

    plsc.subcore_barrier()

    pltpu.sync_copy(sh_cnt.at[pl.ds(base, _ROWS_PER_TILE)],
                    cnt_hbm.at[c, pl.ds(base, _ROWS_PER_TILE)])


def _make_sc_cnt():
    mesh = plsc.VectorSubcoreMesh(core_axis_name="c", subcore_axis_name="s",
                                  num_cores=2, num_subcores=16)
    return pl.kernel(
        _sc_cnt_body,
        out_type=jax.ShapeDtypeStruct((2, _NP, 128), jnp.float32),
        mesh=mesh,
        scratch_types=[
            pltpu.VMEM_SHARED((_NP, 128), jnp.float32),
            pltpu.VMEM((_CHUNKS, 128), jnp.int32),
            pltpu.VMEM((128,), jnp.int32),
            pltpu.VMEM((128, 128), jnp.float32),
        ],
    )


def _mm1(x, wcat):
    def body(x_ref, w_ref, yl_ref, yr_ref):
        y = jnp.dot(x_ref[...].astype(jnp.bfloat16), w_ref[...],
                    preferred_element_type=jnp.float32)
        yl_ref[0] = y[:, :128]
        yl_ref[1] = y[:, 128:256]
        yr_ref[...] = y[:, 256:]

    return pl.pallas_call(
        body,
        grid=(_N // _RB,),
        in_specs=[
            pl.BlockSpec((_RB, _D), lambda i: (i, 0)),
            pl.BlockSpec((_D, 2 * _H), lambda i: (0, 0)),
        ],
        out_specs=[
            pl.BlockSpec((2, _RB, 128), lambda i: (0, i, 0)),
            pl.BlockSpec((_RB, _H), lambda i: (i, 0)),
        ],
        out_shape=[
            jax.ShapeDtypeStruct((2, _N, 128), jnp.float32),
            jax.ShapeDtypeStruct((_N, _H), jnp.float32),
        ],
    )(x, wcat)


def _combine_mm(agg, cnt, yr, b, wcat):
    def body(agg_ref, cnt_ref, yr_ref, b_ref, w_ref, yl_o, yr_o):
        cv = jnp.maximum(cnt_ref[0, :, 0:1] + cnt_ref[1, :, 0:1], 1.0)
        mean = jnp.concatenate([agg_ref[0], agg_ref[1]], axis=1) / cv
        h = jnp.maximum(mean + b_ref[...] + yr_ref[...], 0.0)
        y = jnp.dot(h, w_ref[...], preferred_element_type=jnp.float32)
        yl_o[0] = y[:, :128]
        yl_o[1] = y[:, 128:256]
        yr_o[...] = y[:, 256:]

    return pl.pallas_call(
        body,
        grid=(_N // _RB,),
        in_specs=[
            pl.BlockSpec((2, _RB, 128), lambda i: (0, i, 0)),
            pl.BlockSpec((2, _RB, 128), lambda i: (0, i, 0)),
            pl.BlockSpec((_RB, _H), lambda i: (i, 0)),
            pl.BlockSpec((1, _H), lambda i: (0, 0)),
            pl.BlockSpec((_H, 2 * _H), lambda i: (0, 0)),
        ],
        out_specs=[
            pl.BlockSpec((2, _RB, 128), lambda i: (0, i, 0)),
            pl.BlockSpec((_RB, _H), lambda i: (i, 0)),
        ],
        out_shape=[
            jax.ShapeDtypeStruct((2, _N, 128), jnp.float32),
            jax.ShapeDtypeStruct((_N, _H), jnp.float32),
        ],
    )(agg, cnt, yr, b, wcat)


def _combine_only(agg, cnt, yr, b):
    def body(agg_ref, cnt_ref, yr_ref, b_ref, h_o, hblk_o):
        cv = jnp.maximum(cnt_ref[0, :, 0:1] + cnt_ref[1, :, 0:1], 1.0)
        mean = jnp.concatenate([agg_ref[0], agg_ref[1]], axis=1) / cv
        h = jnp.maximum(mean + b_ref[...] + yr_ref[...], 0.0)
        h_o[...] = h
        hblk_o[0] = h[:, :128]
        hblk_o[1] = h[:, 128:256]

    return pl.pallas_call(
        body,
        grid=(_N // _RB,),
        in_specs=[
            pl.BlockSpec((2, _RB, 128), lambda i: (0, i, 0)),
            pl.BlockSpec((2, _RB, 128), lambda i: (0, i, 0)),
            pl.BlockSpec((_RB, _H), lambda i: (i, 0)),
            pl.BlockSpec((1, _H), lambda i: (0, 0)),
        ],
        out_specs=[
            pl.BlockSpec((_RB, _H), lambda i: (i, 0)),
            pl.BlockSpec((2, _RB, 128), lambda i: (0, i, 0)),
        ],
        out_shape=[
            jax.ShapeDtypeStruct((_N, _H), jnp.float32),
            jax.ShapeDtypeStruct((2, _N, 128), jnp.float32),
        ],
    )(agg, cnt, yr, b)


def _final(agg, cnt, h, wl, wr, b):
    def body(agg_ref, cnt_ref, h_ref, wl_ref, wr_ref, b_ref, o_ref):
        cv = jnp.maximum(cnt_ref[0, :, 0:1] + cnt_ref[1, :, 0:1], 1.0)
        mean = jnp.concatenate([agg_ref[0], agg_ref[1]], axis=1) / cv
        o_ref[...] = (
            jnp.dot(mean, wl_ref[...], preferred_element_type=jnp.float32)
            + b_ref[...]
            + jnp.dot(h_ref[...], wr_ref[...],
                      preferred_element_type=jnp.float32))

    return pl.pallas_call(
        body,
        grid=(_N // _RB,),
        in_specs=[
            pl.BlockSpec((2, _RB, 128), lambda i: (0, i, 0)),
            pl.BlockSpec((2, _RB, 128), lambda i: (0, i, 0)),
            pl.BlockSpec((_RB, _H), lambda i: (i, 0)),
            pl.BlockSpec((_H, _C), lambda i: (0, 0)),
            pl.BlockSpec((_H, _C), lambda i: (0, 0)),
            pl.BlockSpec((1, _C), lambda i: (0, 0)),
        ],
        out_specs=pl.BlockSpec((_RB, _C), lambda i: (i, 0)),
        out_shape=jax.ShapeDtypeStruct((_N, _C), jnp.float32),
    )(agg, cnt, h, wl, wr, b)


_sc_agg = _make_sc_agg()
_sc_cnt = _make_sc_cnt()


def kernel(x, edge_index, W_l1, b_l1, W_r1, W_l2, b_l2, W_r2,
           W_l3, b_l3, W_r3):
    src = edge_index[0]
    dst = edge_index[1]
    pad = _EP - _E
    srcp = jnp.concatenate(
        [src, jnp.zeros((pad,), jnp.int32)]).reshape(16, _CHUNKS, 128)
    dstp = jnp.concatenate(
        [dst, jnp.full((pad,), _N, jnp.int32)]).reshape(16, _CHUNKS, 128)

    wcat1 = jnp.concatenate([W_l1, W_r1], axis=0).T.astype(jnp.bfloat16)
    wcat2 = jnp.concatenate([W_l2, W_r2], axis=0).T

    y1l, y1r = _mm1(x, wcat1)
    cntp = _sc_cnt(dstp)
    agg1 = _sc_agg(y1l.reshape(2 * _N, 128), srcp, dstp)
    y2l, y2r = _combine_mm(agg1, cntp, y1r, b_l1.reshape(1, _H), wcat2)
    agg2 = _sc_agg(y2l.reshape(2 * _N, 128), srcp, dstp)
    h2, h2blk = _combine_only(agg2, cntp, y2r, b_l2.reshape(1, _H))
    agg3 = _sc_agg(h2blk.reshape(2 * _N, 128), srcp, dstp)
    out = _final(agg3, cntp, h2, W_l3.T, W_r3.T, b_l3.reshape(1, _C))
    return (h2, out)
